# Initial kernel scaffold; baseline (speedup 1.0000x reference)
import jax
import jax.numpy as jnp
from jax.experimental import pallas as pl


def kernel(tokens, table):
    raise NotImplementedError("write your pallas kernel here")


# speedup vs baseline: 1.8677x; 1.8677x over previous
import functools

import numpy as np
import jax
import jax.numpy as jnp
from jax import lax
from jax.experimental import pallas as pl
from jax.experimental.pallas import tpu as pltpu
from jax.experimental.pallas import tpu_sc as plsc

_VOCAB = 75
_D = 128
_L = 200
_B = 1024
_PAD = 2

_NC = 2
_NS = 16
_NW = _NC * _NS
_BPW = _B // _NW
_LH = _L // 2


def _pos_encoding() -> np.ndarray:
    even_i = np.arange(0, _D, 2, dtype=np.float32)
    denom = np.power(10000.0, even_i / np.float32(_D))
    pos = np.arange(_L, dtype=np.float32).reshape(_L, 1)
    even = np.sin(pos / denom)
    odd = np.cos(pos / denom)
    return np.stack([even, odd], axis=2).reshape(_L, _D).astype(np.float32)


_MESH = plsc.VectorSubcoreMesh(core_axis_name="c", subcore_axis_name="s")


@functools.partial(
    pl.kernel,
    out_type=jax.ShapeDtypeStruct((_B, _L, _D), jnp.float32),
    mesh=_MESH,
    scratch_types=[
        pltpu.VMEM((2, _LH), jnp.int32),
        pltpu.VMEM((_L, _D), jnp.float32),
        pltpu.VMEM((_L, _D), jnp.float32),
        pltpu.SemaphoreType.DMA,
    ],
)
def _embed(tokens_hbm, table_hbm, pe_hbm, out_hbm, tok_v, rows_v, pe_v, sem):
    wid = lax.axis_index("s") * _NC + lax.axis_index("c")
    pltpu.sync_copy(pe_hbm, pe_v)

    def body(i, carry):
        b = wid * _BPW + i
        pltpu.sync_copy(tokens_hbm.at[b], tok_v)
        c0 = pltpu.async_copy(table_hbm.at[tok_v.at[0]],
                              rows_v.at[pl.ds(0, _LH)], sem)
        c1 = pltpu.async_copy(table_hbm.at[tok_v.at[1]],
                              rows_v.at[pl.ds(_LH, _LH)], sem)
        c0.wait()
        c1.wait()

        def add_pe(r, c):
            for j in range(_D // 16):
                s = pl.ds(16 * j, 16)
                rows_v[r, s] = rows_v[r, s] + pe_v[r, s]
            return c

        lax.fori_loop(0, _L, add_pe, 0)
        pltpu.sync_copy(rows_v, out_hbm.at[b])
        return carry

    lax.fori_loop(0, _BPW, body, 0)


def kernel(tokens, table):
    table = table.at[_PAD].set(0.0)
    pe = jnp.asarray(_pos_encoding())
    tokens3 = tokens.reshape(_B, 2, _LH).astype(jnp.int32)
    return _embed(tokens3, table, pe)


# speedup vs baseline: 1.8987x; 1.0166x over previous
import functools

import numpy as np
import jax
import jax.numpy as jnp
from jax import lax
from jax.experimental import pallas as pl
from jax.experimental.pallas import tpu as pltpu
from jax.experimental.pallas import tpu_sc as plsc

_VOCAB = 75
_D = 128
_L = 200
_B = 1024
_PAD = 2

_NC = 2
_NS = 16
_NW = _NC * _NS
_BPW = _B // _NW
_LH = _L // 2


def _pos_encoding() -> np.ndarray:
    even_i = np.arange(0, _D, 2, dtype=np.float32)
    denom = np.power(10000.0, even_i / np.float32(_D))
    pos = np.arange(_L, dtype=np.float32).reshape(_L, 1)
    even = np.sin(pos / denom)
    odd = np.cos(pos / denom)
    return np.stack([even, odd], axis=2).reshape(_L, _D).astype(np.float32)


_MESH = plsc.VectorSubcoreMesh(core_axis_name="c", subcore_axis_name="s")


@functools.partial(
    pl.kernel,
    out_type=jax.ShapeDtypeStruct((_B, _L, _D), jnp.float32),
    mesh=_MESH,
    scratch_types=[
        pltpu.VMEM((_BPW, 2, _LH), jnp.int32),
        pltpu.VMEM((2, _L, _D), jnp.float32),
        pltpu.VMEM((_L, _D), jnp.float32),
        pltpu.SemaphoreType.DMA,
        pltpu.SemaphoreType.DMA,
        pltpu.SemaphoreType.DMA,
        pltpu.SemaphoreType.DMA,
    ],
)
def _embed(tokens_hbm, table_hbm, pe_hbm, out_hbm,
           tok_v, buf, pe_v, gs0, gs1, os0, os1, ):
    wid = lax.axis_index("s") * _NC + lax.axis_index("c")
    base = wid * _BPW
    gs = (gs0, gs1)
    os_ = (os0, os1)

    pltpu.sync_copy(pe_hbm, pe_v)
    pltpu.sync_copy(tokens_hbm.at[pl.ds(base, _BPW)], tok_v)

    def g_descs(e, p):
        return (
            pltpu.make_async_copy(table_hbm.at[tok_v.at[e, 0]],
                                  buf.at[p, pl.ds(0, _LH)], gs[p]),
            pltpu.make_async_copy(table_hbm.at[tok_v.at[e, 1]],
                                  buf.at[p, pl.ds(_LH, _LH)], gs[p]),
        )

    def o_desc(e, p):
        return pltpu.make_async_copy(buf.at[p], out_hbm.at[base + e], os_[p])

    def add_pe(p):
        @plsc.parallel_loop(0, _L, step=1, unroll=4)
        def _(r):
            for j in range(_D // 16):
                s = pl.ds(16 * j, 16)
                buf[p, r, s] = buf[p, r, s] + pe_v[r, s]

    for d in g_descs(0, 0):
        d.start()

    def body(i, carry):
        e0 = 2 * i
        e1 = 2 * i + 1
        last = (_BPW // 2) - 1

        for d in g_descs(e0, 0):
            d.wait()

        @pl.when(i > 0)
        def _():
            o_desc(e1 - 2, 1).wait()

        for d in g_descs(e1, 1):
            d.start()
        add_pe(0)
        o_desc(e0, 0).start()

        for d in g_descs(e1, 1):
            d.wait()
        add_pe(1)

        @pl.when(i < last)
        def _():
            o_desc(e0, 0).wait()
            for d in g_descs(e0 + 2, 0):
                d.start()

        o_desc(e1, 1).start()
        return carry

    lax.fori_loop(0, _BPW // 2, body, 0)
    o_desc(_BPW - 2, 0).wait()
    o_desc(_BPW - 1, 1).wait()


def kernel(tokens, table):
    table = table.at[_PAD].set(0.0)
    pe = jnp.asarray(_pos_encoding())
    tokens3 = tokens.reshape(_B, 2, _LH).astype(jnp.int32)
    return _embed(tokens3, table, pe)
